# Initial kernel scaffold; baseline (speedup 1.0000x reference)
import jax
import jax.numpy as jnp
from jax.experimental import pallas as pl


def kernel(x, edge_index, W_g1, b_g1, W_g2, b_g2, W_f1, b_f1, W_out, b_out):
    raise NotImplementedError("write your pallas kernel here")


# speedup vs baseline: 7.1149x; 7.1149x over previous
import functools

import jax
import jax.numpy as jnp
from jax import lax
from jax.experimental import pallas as pl
from jax.experimental.pallas import tpu as pltpu
from jax.experimental.pallas import tpu_sc as plsc

N = 10000
E = 160000
D = 256
NP = 10240
EP = 163840
NCH = EP // 128
ROWS_PER_TILE = NP // 16
CH_PER_TILE = NCH // 16
IDXB = 8
DEG_CH_PER_TILE = NCH // 32

def _deg_body(dst_hbm, ones_hbm, zeros_hbm, out_hbm, didx, ones_v, table):
    c = lax.axis_index("c")
    s = lax.axis_index("s")
    r0 = s * ROWS_PER_TILE
    pltpu.sync_copy(zeros_hbm.at[pl.ds(r0, ROWS_PER_TILE)],
                    table.at[pl.ds(r0, ROWS_PER_TILE)])
    pltpu.sync_copy(ones_hbm, ones_v)
    base = c * (NCH // 2) + s * DEG_CH_PER_TILE
    pltpu.sync_copy(dst_hbm.at[pl.ds(base, DEG_CH_PER_TILE)], didx)
    plsc.subcore_barrier()

    def step(j, carry):
        pltpu.sync_copy(ones_v, table.at[didx.at[j]], add=True)
        return carry

    lax.fori_loop(0, DEG_CH_PER_TILE, step, 0)
    plsc.subcore_barrier()
    pltpu.sync_copy(table.at[pl.ds(r0, ROWS_PER_TILE)],
                    out_hbm.at[pl.ds(c * NP + r0, ROWS_PER_TILE)])


def _prop_body(hs_hbm, srcoff_hbm, dst_hbm, zeros_hbm, out_hbm,
               sidx, didx, rows0, rows1, acc, semg, sems):
    c = lax.axis_index("c")
    s = lax.axis_index("s")
    r0 = s * ROWS_PER_TILE
    pltpu.sync_copy(zeros_hbm.at[pl.ds(r0, ROWS_PER_TILE)],
                    acc.at[pl.ds(r0, ROWS_PER_TILE)])
    plsc.subcore_barrier()

    def outer(o, carry):
        ch0 = s * CH_PER_TILE + o * IDXB
        pltpu.sync_copy(srcoff_hbm.at[pl.ds(c * NCH + ch0, IDXB)], sidx)
        pltpu.sync_copy(dst_hbm.at[pl.ds(ch0, IDXB)], didx)

        def pair(i, carry2):
            j0 = 2 * i
            j1 = 2 * i + 1
            pltpu.async_copy(hs_hbm.at[sidx.at[j0]], rows0, semg).wait()
            sc0 = pltpu.async_copy(rows0, acc.at[didx.at[j0]], sems, add=True)
            pltpu.async_copy(hs_hbm.at[sidx.at[j1]], rows1, semg).wait()
            sc0.wait()
            sc1 = pltpu.async_copy(rows1, acc.at[didx.at[j1]], sems, add=True)
            sc1.wait()
            return carry2

        lax.fori_loop(0, IDXB // 2, pair, 0)
        return carry

    lax.fori_loop(0, CH_PER_TILE // IDXB, outer, 0)
    plsc.subcore_barrier()
    pltpu.sync_copy(acc.at[pl.ds(r0, ROWS_PER_TILE)],
                    out_hbm.at[pl.ds(c * NP + r0, ROWS_PER_TILE)])


@functools.cache
def _sc_kernels():
    mesh = plsc.VectorSubcoreMesh(core_axis_name="c", subcore_axis_name="s",
                                  num_cores=2, num_subcores=16)
    deg = pl.kernel(
        _deg_body,
        out_type=jax.ShapeDtypeStruct((2 * NP, 128), jnp.float32),
        mesh=mesh,
        scratch_types=[
            pltpu.VMEM((DEG_CH_PER_TILE, 128), jnp.int32),
            pltpu.VMEM((128, 128), jnp.float32),
            pltpu.VMEM_SHARED((NP, 128), jnp.float32),
        ],
    )
    prop = pl.kernel(
        _prop_body,
        out_type=jax.ShapeDtypeStruct((2 * NP, 128), jnp.float32),
        mesh=mesh,
        scratch_types=[
            pltpu.VMEM((IDXB, 128), jnp.int32),
            pltpu.VMEM((IDXB, 128), jnp.int32),
            pltpu.VMEM((128, 128), jnp.float32),
            pltpu.VMEM((128, 128), jnp.float32),
            pltpu.VMEM_SHARED((NP, 128), jnp.float32),
            pltpu.SemaphoreType.DMA,
            pltpu.SemaphoreType.DMA,
        ],
    )
    return deg, prop


_BN = 512
_G = NP // _BN


def _prep_body(x_ref, w_ref, degp_ref, hs_ref, dis_ref):
    deg = degp_ref[0, :, 0:1] + degp_ref[1, :, 0:1] + 1.0
    dis = lax.rsqrt(deg)
    h = jnp.dot(x_ref[...], w_ref[...], preferred_element_type=jnp.float32)
    hs = h * dis
    hs_ref[0] = hs[:, :128]
    hs_ref[1] = hs[:, 128:]
    dis_ref[...] = dis


def _mid_body(acc_ref, hs_ref, dis_ref, b1_ref, w2_ref, out_ref):
    dis = dis_ref[...]
    p = jnp.concatenate(
        [(acc_ref[0] + hs_ref[0]) * dis, (acc_ref[1] + hs_ref[1]) * dis],
        axis=1) + b1_ref[...]
    z = jnp.maximum(p, 0.0)
    h2 = jnp.dot(z, w2_ref[...], preferred_element_type=jnp.float32)
    hs2 = h2 * dis
    out_ref[0] = hs2[:, :128]
    out_ref[1] = hs2[:, 128:]


def _head_body(acc_ref, hs_ref, dis_ref, b2_ref, wf_ref, bf_ref, wo_ref,
               bo_ref, out_ref):
    dis = dis_ref[...]
    p = jnp.concatenate(
        [(acc_ref[0] + hs_ref[0]) * dis, (acc_ref[1] + hs_ref[1]) * dis],
        axis=1) + b2_ref[...]
    h = jnp.maximum(p, 0.0)
    f = jnp.maximum(
        jnp.dot(h, wf_ref[...], preferred_element_type=jnp.float32)
        + bf_ref[...], 0.0)
    out_ref[...] = (
        jnp.dot(f, wo_ref[...], preferred_element_type=jnp.float32)
        + bo_ref[...])


def _row_spec(shape2):
    return pl.BlockSpec((_BN,) + shape2, lambda i: (i,) + (0,) * len(shape2))


def _full_spec(shape):
    return pl.BlockSpec(shape, lambda i: (0,) * len(shape))


_half_spec = pl.BlockSpec((2, _BN, 128), lambda i: (0, i, 0))

_prep_call = pl.pallas_call(
    _prep_body,
    grid=(_G,),
    in_specs=[
        _row_spec((D,)),
        _full_spec((D, D)),
        pl.BlockSpec((2, _BN, 128), lambda i: (0, i, 0)),
    ],
    out_specs=[_half_spec, _row_spec((1,))],
    out_shape=[
        jax.ShapeDtypeStruct((2, NP, 128), jnp.float32),
        jax.ShapeDtypeStruct((NP, 1), jnp.float32),
    ],
)

_mid_call = pl.pallas_call(
    _mid_body,
    grid=(_G,),
    in_specs=[
        _half_spec,
        _half_spec,
        _row_spec((1,)),
        _full_spec((1, D)),
        _full_spec((D, D)),
    ],
    out_specs=_half_spec,
    out_shape=jax.ShapeDtypeStruct((2, NP, 128), jnp.float32),
)

_head_call = pl.pallas_call(
    _head_body,
    grid=(_G,),
    in_specs=[
        _half_spec,
        _half_spec,
        _row_spec((1,)),
        _full_spec((1, D)),
        _full_spec((D, 128)),
        _full_spec((1, 128)),
        _full_spec((128, 1)),
        _full_spec((1, 1)),
    ],
    out_specs=_row_spec((1,)),
    out_shape=jax.ShapeDtypeStruct((NP, 1), jnp.float32),
)


def kernel(x, edge_index, W_g1, b_g1, W_g2, b_g2, W_f1, b_f1, W_out, b_out):
    xp = jnp.zeros((NP, D), jnp.float32).at[:N].set(x)
    src = edge_index[0]
    dst = edge_index[1]
    padi = jnp.full((EP - E,), NP - 1, jnp.int32)
    srcp = jnp.concatenate([src, padi]).reshape(NCH, 128)
    dstp = jnp.concatenate([dst, padi]).reshape(NCH, 128)
    srcoff = jnp.concatenate([srcp, srcp + NP], axis=0)
    zeros128 = jnp.zeros((NP, 128), jnp.float32)
    ones128 = jnp.ones((128, 128), jnp.float32)

    deg_k, prop_k = _sc_kernels()
    degp = deg_k(dstp, ones128, zeros128).reshape(2, NP, 128)
    hs1, dis = _prep_call(xp, W_g1, degp)
    acc1 = prop_k(hs1.reshape(2 * NP, 128), srcoff, dstp,
                  zeros128).reshape(2, NP, 128)
    hs2 = _mid_call(acc1, hs1, dis, b_g1.reshape(1, D), W_g2)
    acc2 = prop_k(hs2.reshape(2 * NP, 128), srcoff, dstp,
                  zeros128).reshape(2, NP, 128)
    out = _head_call(acc2, hs2, dis, b_g2.reshape(1, D), W_f1,
                     b_f1.reshape(1, 128), W_out, b_out.reshape(1, 1))
    return out[:N]


# speedup vs baseline: 7.5936x; 1.0673x over previous
import functools

import jax
import jax.numpy as jnp
from jax import lax
from jax.experimental import pallas as pl
from jax.experimental.pallas import tpu as pltpu
from jax.experimental.pallas import tpu_sc as plsc

N = 10000
E = 160000
D = 256
NP = 10240
EP = 163840
NCH = EP // 128
ROWS_PER_TILE = NP // 16
CH_PER_TILE = NCH // 16
IDXB = 16
DEG_CH_PER_TILE = NCH // 32

def _deg_body(dst_hbm, ones_hbm, zeros_hbm, out_hbm, didx, ones_v, table):
    c = lax.axis_index("c")
    s = lax.axis_index("s")
    r0 = s * ROWS_PER_TILE
    pltpu.sync_copy(zeros_hbm.at[pl.ds(r0, ROWS_PER_TILE)],
                    table.at[pl.ds(r0, ROWS_PER_TILE)])
    pltpu.sync_copy(ones_hbm, ones_v)
    base = c * (NCH // 2) + s * DEG_CH_PER_TILE
    pltpu.sync_copy(dst_hbm.at[pl.ds(base, DEG_CH_PER_TILE)], didx)
    plsc.subcore_barrier()

    def step(j, carry):
        pltpu.sync_copy(ones_v, table.at[didx.at[j]], add=True)
        return carry

    lax.fori_loop(0, DEG_CH_PER_TILE, step, 0)
    plsc.subcore_barrier()
    pltpu.sync_copy(table.at[pl.ds(r0, ROWS_PER_TILE)],
                    out_hbm.at[pl.ds(c * NP + r0, ROWS_PER_TILE)])


def _prop_body(hs_hbm, srcoff_hbm, dst_hbm, zeros_hbm, out_hbm,
               sidxA, sidxB, didxA, didxB, rows0, rows1, acc,
               semg, sems0, sems1, semi):
    c = lax.axis_index("c")
    s = lax.axis_index("s")
    r0 = s * ROWS_PER_TILE
    pltpu.sync_copy(zeros_hbm.at[pl.ds(r0, ROWS_PER_TILE)],
                    acc.at[pl.ds(r0, ROWS_PER_TILE)])
    base = s * CH_PER_TILE

    def run_batch(si, di):
        gd = pltpu.async_copy(hs_hbm.at[si.at[0]], rows0, semg)
        prev = [None, None]
        for j in range(IDXB):
            rbuf, ssem = (rows0, sems0) if j % 2 == 0 else (rows1, sems1)
            gd.wait()
            sd = pltpu.async_copy(rbuf, acc.at[di.at[j]], ssem, add=True)
            other = (j + 1) % 2
            if prev[other] is not None:
                prev[other].wait()
            if j + 1 < IDXB:
                nbuf = rows1 if j % 2 == 0 else rows0
                gd = pltpu.async_copy(hs_hbm.at[si.at[j + 1]], nbuf, semg)
            prev[j % 2] = sd
        prev[(IDXB - 1) % 2].wait()

    pltpu.sync_copy(srcoff_hbm.at[pl.ds(c * NCH + base, IDXB)], sidxA)
    pltpu.sync_copy(dst_hbm.at[pl.ds(base, IDXB)], didxA)
    plsc.subcore_barrier()

    def outer(o, carry):
        bb = base + 2 * o * IDXB
        l0 = pltpu.async_copy(
            srcoff_hbm.at[pl.ds(c * NCH + bb + IDXB, IDXB)], sidxB, semi)
        l1 = pltpu.async_copy(dst_hbm.at[pl.ds(bb + IDXB, IDXB)], didxB, semi)
        run_batch(sidxA, didxA)
        l0.wait()
        l1.wait()
        l2 = pltpu.async_copy(
            srcoff_hbm.at[pl.ds(c * NCH + bb + 2 * IDXB, IDXB)], sidxA, semi)
        l3 = pltpu.async_copy(dst_hbm.at[pl.ds(bb + 2 * IDXB, IDXB)], didxA,
                              semi)
        run_batch(sidxB, didxB)
        l2.wait()
        l3.wait()
        return carry

    lax.fori_loop(0, (CH_PER_TILE // IDXB) // 2, outer, 0)
    run_batch(sidxA, didxA)
    plsc.subcore_barrier()
    pltpu.sync_copy(acc.at[pl.ds(r0, ROWS_PER_TILE)],
                    out_hbm.at[pl.ds(c * NP + r0, ROWS_PER_TILE)])


@functools.cache
def _sc_kernels():
    mesh = plsc.VectorSubcoreMesh(core_axis_name="c", subcore_axis_name="s",
                                  num_cores=2, num_subcores=16)
    deg = pl.kernel(
        _deg_body,
        out_type=jax.ShapeDtypeStruct((2 * NP, 128), jnp.float32),
        mesh=mesh,
        scratch_types=[
            pltpu.VMEM((DEG_CH_PER_TILE, 128), jnp.int32),
            pltpu.VMEM((128, 128), jnp.float32),
            pltpu.VMEM_SHARED((NP, 128), jnp.float32),
        ],
    )
    prop = pl.kernel(
        _prop_body,
        out_type=jax.ShapeDtypeStruct((2 * NP, 128), jnp.float32),
        mesh=mesh,
        scratch_types=[
            pltpu.VMEM((IDXB, 128), jnp.int32),
            pltpu.VMEM((IDXB, 128), jnp.int32),
            pltpu.VMEM((IDXB, 128), jnp.int32),
            pltpu.VMEM((IDXB, 128), jnp.int32),
            pltpu.VMEM((128, 128), jnp.float32),
            pltpu.VMEM((128, 128), jnp.float32),
            pltpu.VMEM_SHARED((NP, 128), jnp.float32),
            pltpu.SemaphoreType.DMA,
            pltpu.SemaphoreType.DMA,
            pltpu.SemaphoreType.DMA,
            pltpu.SemaphoreType.DMA,
        ],
    )
    return deg, prop


_BN = 512
_G = NP // _BN


def _prep_body(x_ref, w_ref, degp_ref, hs_ref, dis_ref):
    deg = degp_ref[0, :, 0:1] + degp_ref[1, :, 0:1] + 1.0
    dis = lax.rsqrt(deg)
    h = jnp.dot(x_ref[...], w_ref[...], preferred_element_type=jnp.float32)
    hs = h * dis
    hs_ref[0] = hs[:, :128]
    hs_ref[1] = hs[:, 128:]
    dis_ref[...] = dis


def _mid_body(acc_ref, hs_ref, dis_ref, b1_ref, w2_ref, out_ref):
    dis = dis_ref[...]
    p = jnp.concatenate(
        [(acc_ref[0] + hs_ref[0]) * dis, (acc_ref[1] + hs_ref[1]) * dis],
        axis=1) + b1_ref[...]
    z = jnp.maximum(p, 0.0)
    h2 = jnp.dot(z, w2_ref[...], preferred_element_type=jnp.float32)
    hs2 = h2 * dis
    out_ref[0] = hs2[:, :128]
    out_ref[1] = hs2[:, 128:]


def _head_body(acc_ref, hs_ref, dis_ref, b2_ref, wf_ref, bf_ref, wo_ref,
               bo_ref, out_ref):
    dis = dis_ref[...]
    p = jnp.concatenate(
        [(acc_ref[0] + hs_ref[0]) * dis, (acc_ref[1] + hs_ref[1]) * dis],
        axis=1) + b2_ref[...]
    h = jnp.maximum(p, 0.0)
    f = jnp.maximum(
        jnp.dot(h, wf_ref[...], preferred_element_type=jnp.float32)
        + bf_ref[...], 0.0)
    out_ref[...] = (
        jnp.dot(f, wo_ref[...], preferred_element_type=jnp.float32)
        + bo_ref[...])


def _row_spec(shape2):
    return pl.BlockSpec((_BN,) + shape2, lambda i: (i,) + (0,) * len(shape2))


def _full_spec(shape):
    return pl.BlockSpec(shape, lambda i: (0,) * len(shape))


_half_spec = pl.BlockSpec((2, _BN, 128), lambda i: (0, i, 0))

_prep_call = pl.pallas_call(
    _prep_body,
    grid=(_G,),
    in_specs=[
        _row_spec((D,)),
        _full_spec((D, D)),
        pl.BlockSpec((2, _BN, 128), lambda i: (0, i, 0)),
    ],
    out_specs=[_half_spec, _row_spec((1,))],
    out_shape=[
        jax.ShapeDtypeStruct((2, NP, 128), jnp.float32),
        jax.ShapeDtypeStruct((NP, 1), jnp.float32),
    ],
)

_mid_call = pl.pallas_call(
    _mid_body,
    grid=(_G,),
    in_specs=[
        _half_spec,
        _half_spec,
        _row_spec((1,)),
        _full_spec((1, D)),
        _full_spec((D, D)),
    ],
    out_specs=_half_spec,
    out_shape=jax.ShapeDtypeStruct((2, NP, 128), jnp.float32),
)

_head_call = pl.pallas_call(
    _head_body,
    grid=(_G,),
    in_specs=[
        _half_spec,
        _half_spec,
        _row_spec((1,)),
        _full_spec((1, D)),
        _full_spec((D, 128)),
        _full_spec((1, 128)),
        _full_spec((128, 1)),
        _full_spec((1, 1)),
    ],
    out_specs=_row_spec((1,)),
    out_shape=jax.ShapeDtypeStruct((NP, 1), jnp.float32),
)


def kernel(x, edge_index, W_g1, b_g1, W_g2, b_g2, W_f1, b_f1, W_out, b_out):
    xp = jnp.zeros((NP, D), jnp.float32).at[:N].set(x)
    src = edge_index[0]
    dst = edge_index[1]
    padi = jnp.full((EP - E,), NP - 1, jnp.int32)
    srcp = jnp.concatenate([src, padi]).reshape(NCH, 128)
    dstp = jnp.concatenate([dst, padi]).reshape(NCH, 128)
    srcoff = jnp.concatenate([srcp, srcp + NP], axis=0)
    zeros128 = jnp.zeros((NP, 128), jnp.float32)
    ones128 = jnp.ones((128, 128), jnp.float32)

    deg_k, prop_k = _sc_kernels()
    degp = deg_k(dstp, ones128, zeros128).reshape(2, NP, 128)
    hs1, dis = _prep_call(xp, W_g1, degp)
    acc1 = prop_k(hs1.reshape(2 * NP, 128), srcoff, dstp,
                  zeros128).reshape(2, NP, 128)
    hs2 = _mid_call(acc1, hs1, dis, b_g1.reshape(1, D), W_g2)
    acc2 = prop_k(hs2.reshape(2 * NP, 128), srcoff, dstp,
                  zeros128).reshape(2, NP, 128)
    out = _head_call(acc2, hs2, dis, b_g2.reshape(1, D), W_f1,
                     b_f1.reshape(1, 128), W_out, b_out.reshape(1, 1))
    return out[:N]
